# baseline (speedup 1.0000x reference)
import functools

import jax
import jax.numpy as jnp
from jax import lax
from jax.experimental import pallas as pl
from jax.experimental.pallas import tpu as pltpu
from jax.experimental.pallas import tpu_sc as plsc

N = 10000
D = 256
H = 128
NC = 2
NS = 16
GRP = 128
CHUNK_G = 8
CHUNKS_PER_TILE = 10
E_PAD = NS * CHUNKS_PER_TILE * CHUNK_G * GRP
N_TRASH = 240
N_PAD = N + N_TRASH
ZROWS = N_PAD // NS
WROWS = N_PAD // NS


NBUF = 2


def _sc_body(xlo, xhi, epk, agg_lo, agg_hi, deg0_out, deg1_out,
             agg_sh, deg_sh, e2, r0, r1, ones_v, zvec,
             sg0, sg1, ss0, ss1, sdg):
    c = lax.axis_index("c")
    s = lax.axis_index("s")
    rows = (r0, r1)
    sg = (sg0, sg1)
    ss = (ss0, ss1)

    zero16 = jnp.zeros((16,), jnp.float32)
    one16 = jnp.ones((16,), jnp.float32)

    def _zero_row(r, _):
        for j in range(H // 16):
            r0[r, pl.ds(j * 16, 16)] = zero16
        return _

    lax.fori_loop(0, GRP, _zero_row, None)
    for t in range(640 // 16):
        zvec[pl.ds(t * 16, 16)] = zero16
    for j in range(GRP // 16):
        ones_v[pl.ds(j * 16, 16)] = one16

    zbase = s * ZROWS
    for i in range(ZROWS // GRP):
        pltpu.sync_copy(r0, agg_sh.at[pl.ds(zbase + i * GRP, GRP)])
    rem = ZROWS % GRP
    if rem:
        pltpu.sync_copy(r0.at[pl.ds(0, rem)],
                        agg_sh.at[pl.ds(zbase + (ZROWS // GRP) * GRP, rem)])
    pltpu.sync_copy(zvec.at[pl.ds(0, ZROWS)], deg_sh.at[pl.ds(zbase, ZROWS)])
    plsc.subcore_barrier()

    def _edges(x_ref, deg_par):
        def _chunk(i, carry):
            g0 = (s * CHUNKS_PER_TILE + i) * CHUNK_G
            pltpu.sync_copy(epk.at[:, pl.ds(g0, CHUNK_G)], e2)
            deg_d = []
            gat = [pltpu.async_copy(x_ref.at[e2.at[0, j]], rows[j], sg[j])
                   for j in range(NBUF)]
            sca = [None] * NBUF
            for g in range(CHUNK_G):
                j = g % NBUF
                gat[j].wait()
                if g + NBUF < CHUNK_G:
                    sca[j] = pltpu.async_copy(
                        rows[j], agg_sh.at[e2.at[1, g]], ss[j], add=True)
                    if g % 2 == deg_par:
                        deg_d.append(pltpu.async_copy(
                            ones_v, deg_sh.at[e2.at[1, g]], sdg, add=True))
                    sca[j].wait()
                    gat[j] = pltpu.async_copy(
                        x_ref.at[e2.at[0, g + NBUF]], rows[j], sg[j])
                else:
                    sca[j] = pltpu.async_copy(
                        rows[j], agg_sh.at[e2.at[1, g]], ss[j], add=True)
                    if g % 2 == deg_par:
                        deg_d.append(pltpu.async_copy(
                            ones_v, deg_sh.at[e2.at[1, g]], sdg, add=True))
            for j in range(NBUF):
                sca[j].wait()
            for d in deg_d:
                d.wait()
            return carry

        lax.fori_loop(0, CHUNKS_PER_TILE, _chunk, None)

    @pl.when(c == 0)
    def _():
        _edges(xlo, 0)

    @pl.when(c == 1)
    def _():
        _edges(xhi, 1)

    plsc.subcore_barrier()

    wbase = s * WROWS

    @pl.when(c == 0)
    def _():
        pltpu.sync_copy(agg_sh.at[pl.ds(wbase, WROWS)],
                        agg_lo.at[pl.ds(wbase, WROWS)])

    @pl.when(c == 1)
    def _():
        pltpu.sync_copy(agg_sh.at[pl.ds(wbase, WROWS)],
                        agg_hi.at[pl.ds(wbase, WROWS)])

    @pl.when(c == 0)
    def _():
        pltpu.sync_copy(deg_sh.at[pl.ds(s * WROWS, WROWS)],
                        zvec.at[pl.ds(0, WROWS)])
        pltpu.sync_copy(zvec.at[pl.ds(0, WROWS)],
                        deg0_out.at[pl.ds(s * WROWS, WROWS)])

    @pl.when(c == 1)
    def _():
        pltpu.sync_copy(deg_sh.at[pl.ds(s * WROWS, WROWS)],
                        zvec.at[pl.ds(0, WROWS)])
        pltpu.sync_copy(zvec.at[pl.ds(0, WROWS)],
                        deg1_out.at[pl.ds(s * WROWS, WROWS)])


_sc_agg = pl.kernel(
    _sc_body,
    out_type=(
        jax.ShapeDtypeStruct((N_PAD, H), jnp.float32),
        jax.ShapeDtypeStruct((N_PAD, H), jnp.float32),
        jax.ShapeDtypeStruct((N_PAD,), jnp.float32),
        jax.ShapeDtypeStruct((N_PAD,), jnp.float32),
    ),
    mesh=plsc.VectorSubcoreMesh(core_axis_name="c", subcore_axis_name="s",
                                num_cores=NC, num_subcores=NS),
    scratch_types=(
        pltpu.VMEM_SHARED((N_PAD, H), jnp.float32),
        pltpu.VMEM_SHARED((N_PAD,), jnp.float32),
        pltpu.VMEM((2, CHUNK_G, GRP), jnp.int32),
        pltpu.VMEM((GRP, H), jnp.float32),
        pltpu.VMEM((GRP, H), jnp.float32),
        pltpu.VMEM((GRP,), jnp.float32),
        pltpu.VMEM((640,), jnp.float32),
        pltpu.SemaphoreType.DMA,
        pltpu.SemaphoreType.DMA,
        pltpu.SemaphoreType.DMA,
        pltpu.SemaphoreType.DMA,
        pltpu.SemaphoreType.DMA,
    ),
)


def _tc_body(agg_lo, agg_hi, deg0, deg1, w, b, gamma, beta, out):
    d = deg0[...] + deg1[...]
    rt = jnp.transpose(1.0 / jnp.maximum(d, 1.0))
    m = (jnp.dot(agg_lo[...], w[0:H, :], preferred_element_type=jnp.float32)
         + jnp.dot(agg_hi[...], w[H:D, :], preferred_element_type=jnp.float32))
    for k in range(ROWS_BLK // 128):
        h = m[k * 128:(k + 1) * 128, :] * rt[:, k:k + 1] + b[...]
        mu = jnp.mean(h, axis=1, keepdims=True)
        var = jnp.mean((h - mu) * (h - mu), axis=1, keepdims=True)
        hn = (h - mu) * lax.rsqrt(var + 1e-5) * gamma[...] + beta[...]
        out[pl.ds(k * 128, 128), :] = jnp.maximum(hn, 0.0)


ROWS_BLK = 2048


def _tc_tail(agg_lo, agg_hi, deg0, deg1, W, b, gamma, beta):
    grid = N_PAD // ROWS_BLK
    return pl.pallas_call(
        _tc_body,
        grid=(grid,),
        in_specs=[
            pl.BlockSpec((ROWS_BLK, H), lambda i: (i, 0)),
            pl.BlockSpec((ROWS_BLK, H), lambda i: (i, 0)),
            pl.BlockSpec((ROWS_BLK // 128, 128), lambda i: (i, 0)),
            pl.BlockSpec((ROWS_BLK // 128, 128), lambda i: (i, 0)),
            pl.BlockSpec((D, D), lambda i: (0, 0)),
            pl.BlockSpec((1, D), lambda i: (0, 0)),
            pl.BlockSpec((1, D), lambda i: (0, 0)),
            pl.BlockSpec((1, D), lambda i: (0, 0)),
        ],
        out_specs=pl.BlockSpec((ROWS_BLK, D), lambda i: (i, 0)),
        out_shape=jax.ShapeDtypeStruct((N, D), jnp.float32),
    )(agg_lo, agg_hi, deg0, deg1, W, b, gamma, beta)


def kernel(x, edge_index, W, b, gamma, beta):
    xlo = x[:, :H]
    xhi = x[:, H:]
    src = edge_index[0]
    dst = edge_index[1]
    npad = E_PAD - src.shape[0]
    ar = jnp.arange(npad, dtype=jnp.int32)
    pads = jnp.stack([ar % N, N + ar % N_TRASH])
    epk = jnp.concatenate([edge_index, pads], axis=1).reshape(2, E_PAD // GRP, GRP)
    agg_lo, agg_hi, deg0, deg1 = _sc_agg(xlo, xhi, epk)
    return _tc_tail(agg_lo, agg_hi, deg0.reshape(N_PAD // 128, 128),
                    deg1.reshape(N_PAD // 128, 128),
                    W, b.reshape(1, D), gamma.reshape(1, D),
                    beta.reshape(1, D))


# speedup vs baseline: 1.0223x; 1.0223x over previous
import functools

import jax
import jax.numpy as jnp
from jax import lax
from jax.experimental import pallas as pl
from jax.experimental.pallas import tpu as pltpu
from jax.experimental.pallas import tpu_sc as plsc

N = 10000
D = 256
H = 128
NC = 2
NS = 16
GRP = 128
CHUNK_G = 8
CHUNKS_PER_TILE = 10
E_PAD = NS * CHUNKS_PER_TILE * CHUNK_G * GRP
N_TRASH = 240
N_PAD = N + N_TRASH
ZROWS = N_PAD // NS
WROWS = N_PAD // NS


NBUF = 2


def _sc_body(xlo, xhi, epk, agg_lo, agg_hi, deg0_out, deg1_out,
             agg_sh, deg_sh, e2, r0, r1, ones_v, zvec,
             sg0, sg1, ss0, ss1, sdg):
    c = lax.axis_index("c")
    s = lax.axis_index("s")
    rows = (r0, r1)
    sg = (sg0, sg1)
    ss = (ss0, ss1)

    zero16 = jnp.zeros((16,), jnp.float32)
    one16 = jnp.ones((16,), jnp.float32)

    def _zero_row(r, _):
        for j in range(H // 16):
            r0[r, pl.ds(j * 16, 16)] = zero16
        return _

    lax.fori_loop(0, GRP, _zero_row, None)
    for t in range(640 // 16):
        zvec[pl.ds(t * 16, 16)] = zero16
    for j in range(GRP // 16):
        ones_v[pl.ds(j * 16, 16)] = one16

    zbase = s * ZROWS
    for i in range(ZROWS // GRP):
        pltpu.sync_copy(r0, agg_sh.at[pl.ds(zbase + i * GRP, GRP)])
    rem = ZROWS % GRP
    if rem:
        pltpu.sync_copy(r0.at[pl.ds(0, rem)],
                        agg_sh.at[pl.ds(zbase + (ZROWS // GRP) * GRP, rem)])
    pltpu.sync_copy(zvec.at[pl.ds(0, ZROWS)], deg_sh.at[pl.ds(zbase, ZROWS)])
    plsc.subcore_barrier()

    def _edges(x_ref, deg_par):
        def _chunk(i, carry):
            g0 = (s * CHUNKS_PER_TILE + i) * CHUNK_G
            pltpu.sync_copy(epk.at[:, pl.ds(g0, CHUNK_G)], e2)
            deg_d = []
            gat = [pltpu.async_copy(x_ref.at[e2.at[0, j]], rows[j], sg[j])
                   for j in range(NBUF)]
            sca = [None] * NBUF
            for g in range(CHUNK_G):
                j = g % NBUF
                gat[j].wait()
                if g + NBUF < CHUNK_G:
                    sca[j] = pltpu.async_copy(
                        rows[j], agg_sh.at[e2.at[1, g]], ss[j], add=True)
                    if g % 2 == deg_par:
                        deg_d.append(pltpu.async_copy(
                            ones_v, deg_sh.at[e2.at[1, g]], sdg, add=True))
                    sca[j].wait()
                    gat[j] = pltpu.async_copy(
                        x_ref.at[e2.at[0, g + NBUF]], rows[j], sg[j])
                else:
                    sca[j] = pltpu.async_copy(
                        rows[j], agg_sh.at[e2.at[1, g]], ss[j], add=True)
                    if g % 2 == deg_par:
                        deg_d.append(pltpu.async_copy(
                            ones_v, deg_sh.at[e2.at[1, g]], sdg, add=True))
            for j in range(NBUF):
                sca[j].wait()
            for d in deg_d:
                d.wait()
            return carry

        lax.fori_loop(0, CHUNKS_PER_TILE, _chunk, None)

    @pl.when(c == 0)
    def _():
        _edges(xlo, 0)

    @pl.when(c == 1)
    def _():
        _edges(xhi, 1)

    plsc.subcore_barrier()

    wbase = s * WROWS

    @pl.when(c == 0)
    def _():
        pltpu.sync_copy(agg_sh.at[pl.ds(wbase, WROWS)],
                        agg_lo.at[pl.ds(wbase, WROWS)])

    @pl.when(c == 1)
    def _():
        pltpu.sync_copy(agg_sh.at[pl.ds(wbase, WROWS)],
                        agg_hi.at[pl.ds(wbase, WROWS)])

    @pl.when(c == 0)
    def _():
        pltpu.sync_copy(deg_sh.at[pl.ds(s * WROWS, WROWS)],
                        zvec.at[pl.ds(0, WROWS)])
        pltpu.sync_copy(zvec.at[pl.ds(0, WROWS)],
                        deg0_out.at[pl.ds(s * WROWS, WROWS)])

    @pl.when(c == 1)
    def _():
        pltpu.sync_copy(deg_sh.at[pl.ds(s * WROWS, WROWS)],
                        zvec.at[pl.ds(0, WROWS)])
        pltpu.sync_copy(zvec.at[pl.ds(0, WROWS)],
                        deg1_out.at[pl.ds(s * WROWS, WROWS)])


_sc_agg = pl.kernel(
    _sc_body,
    out_type=(
        jax.ShapeDtypeStruct((N_PAD, H), jnp.float32),
        jax.ShapeDtypeStruct((N_PAD, H), jnp.float32),
        jax.ShapeDtypeStruct((N_PAD,), jnp.float32),
        jax.ShapeDtypeStruct((N_PAD,), jnp.float32),
    ),
    mesh=plsc.VectorSubcoreMesh(core_axis_name="c", subcore_axis_name="s",
                                num_cores=NC, num_subcores=NS),
    scratch_types=(
        pltpu.VMEM_SHARED((N_PAD, H), jnp.float32),
        pltpu.VMEM_SHARED((N_PAD,), jnp.float32),
        pltpu.VMEM((2, CHUNK_G, GRP), jnp.int32),
        pltpu.VMEM((GRP, H), jnp.float32),
        pltpu.VMEM((GRP, H), jnp.float32),
        pltpu.VMEM((GRP,), jnp.float32),
        pltpu.VMEM((640,), jnp.float32),
        pltpu.SemaphoreType.DMA,
        pltpu.SemaphoreType.DMA,
        pltpu.SemaphoreType.DMA,
        pltpu.SemaphoreType.DMA,
        pltpu.SemaphoreType.DMA,
    ),
)


def _tc_body(agg_lo, agg_hi, deg0, deg1, w, b, gamma, beta, out):
    d = deg0[...] + deg1[...]
    rt = jnp.transpose(1.0 / jnp.maximum(d, 1.0))
    for k in range(ROWS_BLK // 128):
        rk = rt[:, k:k + 1]
        al = agg_lo[pl.ds(k * 128, 128), :] * rk
        ah = agg_hi[pl.ds(k * 128, 128), :] * rk
        h = (jnp.dot(al, w[0:H, :], preferred_element_type=jnp.float32)
             + jnp.dot(ah, w[H:D, :], preferred_element_type=jnp.float32)
             + b[...])
        mu = jnp.mean(h, axis=1, keepdims=True)
        var = jnp.mean((h - mu) * (h - mu), axis=1, keepdims=True)
        hn = (h - mu) * lax.rsqrt(var + 1e-5) * gamma[...] + beta[...]
        out[pl.ds(k * 128, 128), :] = jnp.maximum(hn, 0.0)


ROWS_BLK = 1024


def _tc_tail(agg_lo, agg_hi, deg0, deg1, W, b, gamma, beta):
    grid = N_PAD // ROWS_BLK
    return pl.pallas_call(
        _tc_body,
        grid=(grid,),
        in_specs=[
            pl.BlockSpec((ROWS_BLK, H), lambda i: (i, 0)),
            pl.BlockSpec((ROWS_BLK, H), lambda i: (i, 0)),
            pl.BlockSpec((ROWS_BLK // 128, 128), lambda i: (i, 0)),
            pl.BlockSpec((ROWS_BLK // 128, 128), lambda i: (i, 0)),
            pl.BlockSpec((D, D), lambda i: (0, 0)),
            pl.BlockSpec((1, D), lambda i: (0, 0)),
            pl.BlockSpec((1, D), lambda i: (0, 0)),
            pl.BlockSpec((1, D), lambda i: (0, 0)),
        ],
        out_specs=pl.BlockSpec((ROWS_BLK, D), lambda i: (i, 0)),
        out_shape=jax.ShapeDtypeStruct((N, D), jnp.float32),
    )(agg_lo, agg_hi, deg0, deg1, W, b, gamma, beta)


def kernel(x, edge_index, W, b, gamma, beta):
    xlo = x[:, :H]
    xhi = x[:, H:]
    src = edge_index[0]
    dst = edge_index[1]
    npad = E_PAD - src.shape[0]
    ar = jnp.arange(npad, dtype=jnp.int32)
    pads = jnp.stack([ar % N, N + ar % N_TRASH])
    epk = jnp.concatenate([edge_index, pads], axis=1).reshape(2, E_PAD // GRP, GRP)
    agg_lo, agg_hi, deg0, deg1 = _sc_agg(xlo, xhi, epk)
    return _tc_tail(agg_lo, agg_hi, deg0.reshape(N_PAD // 128, 128),
                    deg1.reshape(N_PAD // 128, 128),
                    W, b.reshape(1, D), gamma.reshape(1, D),
                    beta.reshape(1, D))
